# Initial kernel scaffold; baseline (speedup 1.0000x reference)
import jax
import jax.numpy as jnp
from jax.experimental import pallas as pl


def kernel(stacked_qc, graph_sizes, w1, b1, w2, b2):
    raise NotImplementedError("write your pallas kernel here")


# speedup vs baseline: 1.5221x; 1.5221x over previous
import functools

import jax
import jax.numpy as jnp
from jax.experimental import pallas as pl
from jax.experimental.pallas import tpu as pltpu


def _lse_lane_tiles(la, n_nodes, bb):
    mx = la[:, 0:bb]
    for i in range(1, n_nodes):
        mx = jnp.maximum(mx, la[:, i * bb:(i + 1) * bb])
    ex = jnp.exp(la - pltpu.repeat(mx, n_nodes, axis=1))
    sm = ex[:, 0:bb]
    for i in range(1, n_nodes):
        sm = sm + ex[:, i * bb:(i + 1) * bb]
    return pltpu.repeat(mx + jnp.log(sm), n_nodes, axis=1)


def _lse_sublanes(la):
    mx = jnp.max(la, axis=0, keepdims=True)
    ex = jnp.exp(la - mx)
    sm = jnp.sum(ex, axis=0, keepdims=True)
    return mx + jnp.log(sm)


def _nanl_kernel(x_ref, m_ref, w1_ref, b1_ref, w2_ref, b2_ref, out_ref,
                 *, n_nodes, bb, inv_temp, sinkhorn_iters):
    N = n_nodes
    L = N * bb
    X = x_ref[0]

    h = jnp.maximum(
        jnp.dot(w1_ref[...], X, preferred_element_type=jnp.float32) + b1_ref[...], 0.0)
    e = jnp.dot(w2_ref[...], h, preferred_element_type=jnp.float32) + b2_ref[...]
    e = e * m_ref[0]

    ce = e[:, L:]
    rows = []
    for n in range(N):
        qr = pltpu.repeat(e[:, n * bb:(n + 1) * bb], N, axis=1)
        rows.append(jnp.sum(qr * ce, axis=0, keepdims=True))
    la = jnp.concatenate(rows, axis=0) * inv_temp

    for _ in range(sinkhorn_iters):
        la = la - _lse_lane_tiles(la, N, bb)
        la = la - _lse_sublanes(la)
    plan = jnp.exp(la)

    sc = jnp.zeros((1, bb), jnp.float32)
    for n in range(N):
        pn = plan[n:n + 1, :]
        recon = pn[:, 0:bb] * X[:, L:L + bb]
        for m in range(1, N):
            recon = recon + pn[:, m * bb:(m + 1) * bb] * X[:, L + m * bb:L + (m + 1) * bb]
        diff = jnp.maximum(X[:, n * bb:(n + 1) * bb] - recon, 0.0)
        sc = sc + jnp.sum(diff, axis=0, keepdims=True)
    out_ref[...] = (-sc).reshape(1, 1, bb)


def _pick_batch_block(batch):
    for d in (128, 64, 32, 16, 8, 4, 2):
        if d <= batch and batch % d == 0 and batch // d >= 2:
            return d
    return batch


def kernel(stacked_qc, graph_sizes, w1, b1, w2, b2):
    B, two, N, F = stacked_qc.shape
    assert two == 2
    T = w1.shape[1]
    bb = _pick_batch_block(B)
    G = B // bb
    L = N * bb

    x = (stacked_qc.astype(jnp.float32)
         .reshape(G, bb, 2, N, F)
         .transpose(0, 4, 2, 3, 1)
         .reshape(G, F, 2 * L))
    ar = jnp.arange(N, dtype=jnp.float32)
    msk = (ar[None, None, :] < graph_sizes.astype(jnp.float32)[:, :, None])
    msk = (msk.astype(jnp.float32)
           .reshape(G, bb, 2, N)
           .transpose(0, 2, 3, 1)
           .reshape(G, 1, 2 * L))

    kern = functools.partial(_nanl_kernel, n_nodes=N, bb=bb,
                             inv_temp=10.0, sinkhorn_iters=10)
    out = pl.pallas_call(
        kern,
        grid=(G,),
        out_shape=jax.ShapeDtypeStruct((G, 1, bb), jnp.float32),
        in_specs=[
            pl.BlockSpec((1, F, 2 * L), lambda g: (g, 0, 0)),
            pl.BlockSpec((1, 1, 2 * L), lambda g: (g, 0, 0)),
            pl.BlockSpec((T, F), lambda g: (0, 0)),
            pl.BlockSpec((T, 1), lambda g: (0, 0)),
            pl.BlockSpec((T, T), lambda g: (0, 0)),
            pl.BlockSpec((T, 1), lambda g: (0, 0)),
        ],
        out_specs=pl.BlockSpec((1, 1, bb), lambda g: (g, 0, 0)),
        compiler_params=pltpu.CompilerParams(
            dimension_semantics=("parallel",),
            vmem_limit_bytes=32 * 1024 * 1024),
    )(x, msk,
      w1.T.astype(jnp.float32), b1.reshape(-1, 1).astype(jnp.float32),
      w2.T.astype(jnp.float32), b2.reshape(-1, 1).astype(jnp.float32))
    return out.reshape(B)


# speedup vs baseline: 2.4820x; 1.6306x over previous
import functools

import jax
import jax.numpy as jnp
from jax.experimental import pallas as pl
from jax.experimental.pallas import tpu as pltpu


def _lse_lane_tiles(la, n_nodes, bb):
    mx = la[:, 0:bb]
    for i in range(1, n_nodes):
        mx = jnp.maximum(mx, la[:, i * bb:(i + 1) * bb])
    ex = jnp.exp(la - pltpu.repeat(mx, n_nodes, axis=1))
    sm = ex[:, 0:bb]
    for i in range(1, n_nodes):
        sm = sm + ex[:, i * bb:(i + 1) * bb]
    return pltpu.repeat(mx + jnp.log(sm), n_nodes, axis=1)


def _lse_sublanes(la):
    mx = jnp.max(la, axis=0, keepdims=True)
    ex = jnp.exp(la - mx)
    sm = jnp.sum(ex, axis=0, keepdims=True)
    return mx + jnp.log(sm)


def _nanl_kernel(x_ref, sz_ref, w1_ref, b1_ref, w2_ref, b2_ref, out_ref,
                 *, n_nodes, bb, inv_temp, sinkhorn_iters):
    N = n_nodes
    L = N * bb

    parts = []
    for s in range(2):
        for n in range(N):
            parts.append(x_ref[0, :, s, n, :].T)
    X = jnp.concatenate(parts, axis=1)

    mparts = []
    for s in range(2):
        szs = sz_ref[0, s:s + 1, :]
        for n in range(N):
            mparts.append(jnp.where(float(n) < szs, 1.0, 0.0))
    mrow = jnp.concatenate(mparts, axis=1)

    h = jnp.maximum(
        jnp.dot(w1_ref[...], X, preferred_element_type=jnp.float32) + b1_ref[...], 0.0)
    e = jnp.dot(w2_ref[...], h, preferred_element_type=jnp.float32) + b2_ref[...]
    e = e * mrow

    ce = e[:, L:]
    rows = []
    for n in range(N):
        qr = pltpu.repeat(e[:, n * bb:(n + 1) * bb], N, axis=1)
        rows.append(jnp.sum(qr * ce, axis=0, keepdims=True))
    la = jnp.concatenate(rows, axis=0) * inv_temp

    for _ in range(sinkhorn_iters):
        la = la - _lse_lane_tiles(la, N, bb)
        la = la - _lse_sublanes(la)
    plan = jnp.exp(la)

    sc = jnp.zeros((1, bb), jnp.float32)
    for n in range(N):
        pn = plan[n:n + 1, :]
        recon = pn[:, 0:bb] * X[:, L:L + bb]
        for m in range(1, N):
            recon = recon + pn[:, m * bb:(m + 1) * bb] * X[:, L + m * bb:L + (m + 1) * bb]
        diff = jnp.maximum(X[:, n * bb:(n + 1) * bb] - recon, 0.0)
        sc = sc + jnp.sum(diff, axis=0, keepdims=True)
    out_ref[...] = (-sc).reshape(1, 1, bb)


def _pick_batch_block(batch):
    for d in (128, 64, 32, 16, 8, 4, 2):
        if d <= batch and batch % d == 0 and batch // d >= 2:
            return d
    return batch


def kernel(stacked_qc, graph_sizes, w1, b1, w2, b2):
    B, two, N, F = stacked_qc.shape
    assert two == 2
    T = w1.shape[1]
    bb = _pick_batch_block(B)
    G = B // bb

    x = stacked_qc.astype(jnp.float32).reshape(G, bb, 2, N, F)
    sz = (graph_sizes.astype(jnp.float32)
          .reshape(G, bb, 2)
          .transpose(0, 2, 1))

    kern = functools.partial(_nanl_kernel, n_nodes=N, bb=bb,
                             inv_temp=10.0, sinkhorn_iters=10)
    out = pl.pallas_call(
        kern,
        grid=(G,),
        out_shape=jax.ShapeDtypeStruct((G, 1, bb), jnp.float32),
        in_specs=[
            pl.BlockSpec((1, bb, 2, N, F), lambda g: (g, 0, 0, 0, 0)),
            pl.BlockSpec((1, 2, bb), lambda g: (g, 0, 0)),
            pl.BlockSpec((T, F), lambda g: (0, 0)),
            pl.BlockSpec((T, 1), lambda g: (0, 0)),
            pl.BlockSpec((T, T), lambda g: (0, 0)),
            pl.BlockSpec((T, 1), lambda g: (0, 0)),
        ],
        out_specs=pl.BlockSpec((1, 1, bb), lambda g: (g, 0, 0)),
        compiler_params=pltpu.CompilerParams(
            dimension_semantics=("parallel",),
            vmem_limit_bytes=32 * 1024 * 1024),
    )(x, sz,
      w1.T.astype(jnp.float32), b1.reshape(-1, 1).astype(jnp.float32),
      w2.T.astype(jnp.float32), b2.reshape(-1, 1).astype(jnp.float32))
    return out.reshape(B)
